# baseline (speedup 1.0000x reference)
import jax
import jax.numpy as jnp
from jax import lax
from jax.experimental import pallas as pl
from jax.experimental.pallas import tpu as pltpu
from jax.experimental.pallas import tpu_sc as plsc

K = 8
VOCAB = 4
HIDDEN = 128
BATCH = 4
SEQ = 8192
NPOS = BATCH * SEQ
NW = 32
POS_PER_W = NPOS // NW
CHUNK = 64
NCHUNK = POS_PER_W // CHUNK
W_PER_ROW = SEQ // POS_PER_W

PAD = 128


NBUF = 4


def _sc_body(ids_hbm, table_hbm, out_hbm, row_v, idx_refs, bufs, gsems, osems):
    cid = lax.axis_index("c")
    sid = lax.axis_index("s")
    wid = sid * 2 + cid
    row = wid // W_PER_ROW
    base = (wid % W_PER_ROW) * POS_PER_W

    z = jnp.zeros((16,), jnp.int32)
    for zo in range(0, PAD, 16):
        row_v[pl.ds(zo, 16)] = z
    row_v[pl.ds(PAD + SEQ, 16)] = z
    pltpu.sync_copy(ids_hbm.at[row], row_v.at[pl.ds(PAD, SEQ)])

    def _compute_idx(c):
        def ibody(i, carry):
            o = base + c * CHUNK + i * 16 + PAD - 3
            acc = row_v[pl.ds(o, 16)]
            for j in range(1, K):
                acc = acc * 4 + row_v[pl.ds(o + j, 16)]
            idx_refs[c][pl.ds(i * 16, 16)] = acc
            return carry

        lax.fori_loop(0, CHUNK // 16, ibody, 0)

    def _gather(c):
        return pltpu.async_copy(
            table_hbm.at[idx_refs[c]], bufs[c % NBUF], gsems[c % NBUF]
        )

    def _writeback(c):
        return pltpu.async_copy(
            bufs[c % NBUF],
            out_hbm.at[pl.ds(row * SEQ + base + c * CHUNK, CHUNK)],
            osems[c % NBUF],
        )

    gh = [None] * NCHUNK
    oh = [None] * NCHUNK
    for c in range(NCHUNK):
        if c >= NBUF:
            oh[c - NBUF].wait()
        _compute_idx(c)
        gh[c] = _gather(c)
        if c >= 1:
            gh[c - 1].wait()
            oh[c - 1] = _writeback(c - 1)
    gh[NCHUNK - 1].wait()
    oh[NCHUNK - 1] = _writeback(NCHUNK - 1)
    for c in range(NCHUNK - NBUF, NCHUNK):
        oh[c].wait()


@jax.jit
def _kmer_embed(input_ids, table):
    mesh = plsc.VectorSubcoreMesh(core_axis_name="c", subcore_axis_name="s")
    k = pl.kernel(
        _sc_body,
        out_type=jax.ShapeDtypeStruct((NPOS, HIDDEN), jnp.float32),
        mesh=mesh,
        scratch_types=[
            pltpu.VMEM((PAD + SEQ + 16,), jnp.int32),
            [pltpu.VMEM((CHUNK,), jnp.int32) for _ in range(NCHUNK)],
            [pltpu.VMEM((CHUNK, HIDDEN), jnp.float32) for _ in range(NBUF)],
            [pltpu.SemaphoreType.DMA for _ in range(NBUF)],
            [pltpu.SemaphoreType.DMA for _ in range(NBUF)],
        ],
    )
    flat = k(input_ids, table)
    return flat.reshape(BATCH, SEQ, HIDDEN)


def kernel(input_ids, table):
    return _kmer_embed(input_ids, table)


# speedup vs baseline: 1.0945x; 1.0945x over previous
import jax
import jax.numpy as jnp
from jax import lax
from jax.experimental import pallas as pl
from jax.experimental.pallas import tpu as pltpu
from jax.experimental.pallas import tpu_sc as plsc

K = 8
VOCAB = 4
HIDDEN = 128
BATCH = 4
SEQ = 8192
NPOS = BATCH * SEQ
NW = 32
POS_PER_W = NPOS // NW
CHUNK = 128
NCHUNK = POS_PER_W // CHUNK
W_PER_ROW = SEQ // POS_PER_W

PAD = 128


NBUF = 3


def _sc_body(ids_hbm, table_hbm, out_hbm, row_v, idx_refs, bufs, gsems, osems):
    cid = lax.axis_index("c")
    sid = lax.axis_index("s")
    wid = sid * 2 + cid
    row = wid // W_PER_ROW
    base = (wid % W_PER_ROW) * POS_PER_W

    z = jnp.zeros((16,), jnp.int32)
    for zo in range(0, PAD, 16):
        row_v[pl.ds(zo, 16)] = z
    row_v[pl.ds(PAD + SEQ, 16)] = z
    pltpu.sync_copy(ids_hbm.at[row], row_v.at[pl.ds(PAD, SEQ)])

    def _compute_idx(c):
        @plsc.parallel_loop(0, CHUNK, step=16)
        def _ibody(t):
            o = base + c * CHUNK + t + PAD - 3
            acc = row_v[pl.ds(o, 16)]
            for j in range(1, K):
                acc = acc * 4 + row_v[pl.ds(o + j, 16)]
            idx_refs[c][pl.ds(t, 16)] = acc

    def _gather(c):
        return pltpu.async_copy(
            table_hbm.at[idx_refs[c]], bufs[c % NBUF], gsems[c % NBUF]
        )

    def _writeback(c):
        return pltpu.async_copy(
            bufs[c % NBUF],
            out_hbm.at[pl.ds(row * SEQ + base + c * CHUNK, CHUNK)],
            osems[c % NBUF],
        )

    gh = [None] * NCHUNK
    oh = [None] * NCHUNK
    for c in range(NCHUNK):
        if c >= NBUF:
            oh[c - NBUF].wait()
        _compute_idx(c)
        gh[c] = _gather(c)
        if c >= 1:
            gh[c - 1].wait()
            oh[c - 1] = _writeback(c - 1)
    gh[NCHUNK - 1].wait()
    oh[NCHUNK - 1] = _writeback(NCHUNK - 1)
    for c in range(NCHUNK - NBUF, NCHUNK):
        oh[c].wait()


@jax.jit
def _kmer_embed(input_ids, table):
    mesh = plsc.VectorSubcoreMesh(core_axis_name="c", subcore_axis_name="s")
    k = pl.kernel(
        _sc_body,
        out_type=jax.ShapeDtypeStruct((NPOS, HIDDEN), jnp.float32),
        mesh=mesh,
        scratch_types=[
            pltpu.VMEM((PAD + SEQ + 16,), jnp.int32),
            [pltpu.VMEM((CHUNK,), jnp.int32) for _ in range(NCHUNK)],
            [pltpu.VMEM((CHUNK, HIDDEN), jnp.float32) for _ in range(NBUF)],
            [pltpu.SemaphoreType.DMA for _ in range(NBUF)],
            [pltpu.SemaphoreType.DMA for _ in range(NBUF)],
        ],
    )
    flat = k(input_ids, table)
    return flat.reshape(BATCH, SEQ, HIDDEN)


def kernel(input_ids, table):
    return _kmer_embed(input_ids, table)


# speedup vs baseline: 1.0994x; 1.0045x over previous
import jax
import jax.numpy as jnp
from jax import lax
from jax.experimental import pallas as pl
from jax.experimental.pallas import tpu as pltpu
from jax.experimental.pallas import tpu_sc as plsc

K = 8
VOCAB = 4
HIDDEN = 128
BATCH = 4
SEQ = 8192
NPOS = BATCH * SEQ
NW = 32
POS_PER_W = NPOS // NW
CHUNK = 128
NCHUNK = POS_PER_W // CHUNK
W_PER_ROW = SEQ // POS_PER_W

PAD = 128


NBUF = 3


def _sc_body(ids_hbm, table_hbm, out_hbm, row_v, idx_refs, bufs, gsems, osems):
    cid = lax.axis_index("c")
    sid = lax.axis_index("s")
    wid = sid * 2 + cid
    row = wid // W_PER_ROW
    base = (wid % W_PER_ROW) * POS_PER_W

    z = jnp.zeros((16,), jnp.int32)
    for zo in range(0, PAD, 16):
        row_v[pl.ds(zo, 16)] = z
    row_v[pl.ds(PAD + SEQ, 16)] = z
    pltpu.sync_copy(ids_hbm.at[row], row_v.at[pl.ds(PAD, SEQ)])

    @plsc.parallel_loop(0, POS_PER_W, step=16)
    def _ibody(t):
        o = base + t + PAD - 3
        acc = row_v[pl.ds(o, 16)]
        for j in range(1, K):
            acc = acc * 4 + row_v[pl.ds(o + j, 16)]
        idx_refs[pl.ds(t, 16)] = acc

    def _gather(c):
        return pltpu.async_copy(
            table_hbm.at[idx_refs.at[pl.ds(c * CHUNK, CHUNK)]],
            bufs[c % NBUF], gsems[c % NBUF],
        )

    def _writeback(c):
        return pltpu.async_copy(
            bufs[c % NBUF],
            out_hbm.at[pl.ds(row * SEQ + base + c * CHUNK, CHUNK)],
            osems[c % NBUF],
        )

    gh = [None] * NCHUNK
    oh = [None] * NCHUNK
    for c in range(NCHUNK):
        if c >= NBUF:
            oh[c - NBUF].wait()
        gh[c] = _gather(c)
        if c >= 1:
            gh[c - 1].wait()
            oh[c - 1] = _writeback(c - 1)
    gh[NCHUNK - 1].wait()
    oh[NCHUNK - 1] = _writeback(NCHUNK - 1)
    for c in range(NCHUNK - NBUF, NCHUNK):
        oh[c].wait()


@jax.jit
def _kmer_embed(input_ids, table):
    mesh = plsc.VectorSubcoreMesh(core_axis_name="c", subcore_axis_name="s")
    k = pl.kernel(
        _sc_body,
        out_type=jax.ShapeDtypeStruct((NPOS, HIDDEN), jnp.float32),
        mesh=mesh,
        scratch_types=[
            pltpu.VMEM((PAD + SEQ + 16,), jnp.int32),
            pltpu.VMEM((POS_PER_W,), jnp.int32),
            [pltpu.VMEM((CHUNK, HIDDEN), jnp.float32) for _ in range(NBUF)],
            [pltpu.SemaphoreType.DMA for _ in range(NBUF)],
            [pltpu.SemaphoreType.DMA for _ in range(NBUF)],
        ],
    )
    flat = k(input_ids, table)
    return flat.reshape(BATCH, SEQ, HIDDEN)


def kernel(input_ids, table):
    return _kmer_embed(input_ids, table)


# speedup vs baseline: 1.1102x; 1.0099x over previous
import jax
import jax.numpy as jnp
from jax import lax
from jax.experimental import pallas as pl
from jax.experimental.pallas import tpu as pltpu
from jax.experimental.pallas import tpu_sc as plsc

K = 8
VOCAB = 4
HIDDEN = 128
BATCH = 4
SEQ = 8192
NPOS = BATCH * SEQ
NW = 32
POS_PER_W = NPOS // NW
CHUNK = 128
NCHUNK = POS_PER_W // CHUNK
W_PER_ROW = SEQ // POS_PER_W

PAD = 128


NBUF = 3


def _sc_body(ids_hbm, table_hbm, out_hbm, row_v, idx_refs, bufs, gsems, osems):
    cid = lax.axis_index("c")
    sid = lax.axis_index("s")
    wid = sid * 2 + cid
    row = wid // W_PER_ROW
    base = (wid % W_PER_ROW) * POS_PER_W

    z = jnp.zeros((16,), jnp.int32)
    for zo in range(0, PAD, 16):
        row_v[pl.ds(zo, 16)] = z
    row_v[pl.ds(PAD + SEQ, 16)] = z
    pltpu.sync_copy(ids_hbm.at[row], row_v.at[pl.ds(PAD, SEQ)])

    def _compute_idx(lo, hi):
        @plsc.parallel_loop(lo, hi, step=16)
        def _ibody(t):
            o = base + t + PAD - 3
            acc = row_v[pl.ds(o, 16)]
            for j in range(1, K):
                acc = acc * 4 + row_v[pl.ds(o + j, 16)]
            idx_refs[pl.ds(t, 16)] = acc

    def _gather(c):
        return pltpu.async_copy(
            table_hbm.at[idx_refs.at[pl.ds(c * CHUNK, CHUNK)]],
            bufs[c % NBUF], gsems[c % NBUF],
        )

    def _writeback(c):
        return pltpu.async_copy(
            bufs[c % NBUF],
            out_hbm.at[pl.ds(row * SEQ + base + c * CHUNK, CHUNK)],
            osems[c % NBUF],
        )

    gh = [None] * NCHUNK
    oh = [None] * NCHUNK
    _compute_idx(0, CHUNK)
    for c in range(NCHUNK):
        if c >= NBUF:
            oh[c - NBUF].wait()
        gh[c] = _gather(c)
        if c == 0:
            _compute_idx(CHUNK, POS_PER_W)
        if c >= 1:
            gh[c - 1].wait()
            oh[c - 1] = _writeback(c - 1)
    gh[NCHUNK - 1].wait()
    oh[NCHUNK - 1] = _writeback(NCHUNK - 1)
    for c in range(NCHUNK - NBUF, NCHUNK):
        oh[c].wait()


@jax.jit
def _kmer_embed(input_ids, table):
    mesh = plsc.VectorSubcoreMesh(core_axis_name="c", subcore_axis_name="s")
    k = pl.kernel(
        _sc_body,
        out_type=jax.ShapeDtypeStruct((NPOS, HIDDEN), jnp.float32),
        mesh=mesh,
        scratch_types=[
            pltpu.VMEM((PAD + SEQ + 16,), jnp.int32),
            pltpu.VMEM((POS_PER_W,), jnp.int32),
            [pltpu.VMEM((CHUNK, HIDDEN), jnp.float32) for _ in range(NBUF)],
            [pltpu.SemaphoreType.DMA for _ in range(NBUF)],
            [pltpu.SemaphoreType.DMA for _ in range(NBUF)],
        ],
    )
    flat = k(input_ids, table)
    return flat.reshape(BATCH, SEQ, HIDDEN)


def kernel(input_ids, table):
    return _kmer_embed(input_ids, table)
